# baseline (speedup 1.0000x reference)
import functools

import jax
import jax.numpy as jnp
from jax import lax
from jax.experimental import pallas as pl
from jax.experimental.pallas import tpu as pltpu

E = 128
V = 128
F = 7
T = 15
SCALE = 1.0 / (E ** 0.5)


def _body(x_ref, w_ref, ft_ref, o_ref, mw_ref):
    @pl.when(pl.program_id(0) == 0)
    def _compose():
        ft = ft_ref[...]
        t_row = lax.broadcasted_iota(jnp.int32, (V, T), 1)
        m = jnp.zeros((V, T), jnp.float32)
        for j in range(F):
            m = m + (ft[:, j:j + 1] == t_row).astype(jnp.float32)
        row0 = (lax.broadcasted_iota(jnp.int32, (1, T), 1) == 0).astype(jnp.float32)
        m_full = jnp.concatenate([row0, m], axis=0)
        mw_ref[...] = lax.dot_general(m_full, w_ref[...],
                                      (((1,), (0,)), ((), ())),
                                      preferred_element_type=jnp.float32) * SCALE

    o_ref[...] = lax.dot_general(x_ref[...], mw_ref[...],
                                 (((1,), (1,)), ((), ())),
                                 preferred_element_type=jnp.float32)


@jax.jit
def kernel(inputs, weight, feature_table):
    B = inputs.shape[0]
    blk = 512
    grid = (B // blk,)
    return pl.pallas_call(
        _body,
        grid=grid,
        in_specs=[
            pl.BlockSpec((blk, E), lambda i: (i, 0)),
            pl.BlockSpec((T, E), lambda i: (0, 0)),
            pl.BlockSpec((V, F), lambda i: (0, 0)),
        ],
        out_specs=pl.BlockSpec((blk, V + 1), lambda i: (i, 0)),
        out_shape=jax.ShapeDtypeStruct((B, V + 1), jnp.float32),
        scratch_shapes=[pltpu.VMEM((V + 1, E), jnp.float32)],
    )(inputs, weight, feature_table)


# speedup vs baseline: 1.5892x; 1.5892x over previous
import functools

import jax
import jax.numpy as jnp
from jax import lax
from jax.experimental import pallas as pl
from jax.experimental.pallas import tpu as pltpu

E = 128
V = 128
F = 7
T = 15
SCALE = 1.0 / (E ** 0.5)


def _body(x_ref, w_ref, ft_ref, o_ref, mw_ref):
    @pl.when(pl.program_id(0) == 0)
    def _compose():
        ft = ft_ref[...]
        t_row = lax.broadcasted_iota(jnp.int32, (V, T), 1)
        m = jnp.zeros((V, T), jnp.float32)
        for j in range(F):
            m = m + (ft[:, j:j + 1] == t_row).astype(jnp.float32)
        row0 = (lax.broadcasted_iota(jnp.int32, (1, T), 1) == 0).astype(jnp.float32)
        m_full = jnp.concatenate([row0, m], axis=0)
        mw_ref[...] = lax.dot_general(m_full, w_ref[...],
                                      (((1,), (0,)), ((), ())),
                                      preferred_element_type=jnp.float32) * SCALE

    o_ref[...] = lax.dot_general(x_ref[...], mw_ref[...],
                                 (((1,), (1,)), ((), ())),
                                 preferred_element_type=jnp.float32)


@jax.jit
def kernel(inputs, weight, feature_table):
    B = inputs.shape[0]
    blk = 4096
    grid = (B // blk,)
    return pl.pallas_call(
        _body,
        grid=grid,
        in_specs=[
            pl.BlockSpec((blk, E), lambda i: (i, 0)),
            pl.BlockSpec((T, E), lambda i: (0, 0)),
            pl.BlockSpec((V, F), lambda i: (0, 0)),
        ],
        out_specs=pl.BlockSpec((blk, V + 1), lambda i: (i, 0)),
        out_shape=jax.ShapeDtypeStruct((B, V + 1), jnp.float32),
        scratch_shapes=[pltpu.VMEM((V + 1, E), jnp.float32)],
    )(inputs, weight, feature_table)


# speedup vs baseline: 1.6262x; 1.0233x over previous
import functools

import jax
import jax.numpy as jnp
from jax import lax
from jax.experimental import pallas as pl
from jax.experimental.pallas import tpu as pltpu

E = 128
V = 128
F = 7
T = 15
SCALE = 1.0 / (E ** 0.5)


def _body(x_ref, w_ref, ft_ref, o_ref, mw_ref):
    @pl.when(pl.program_id(0) == 0)
    def _compose():
        ft = ft_ref[...]
        t_row = lax.broadcasted_iota(jnp.int32, (V, T), 1)
        m = jnp.zeros((V, T), jnp.float32)
        for j in range(F):
            m = m + (ft[:, j:j + 1] == t_row).astype(jnp.float32)
        row0 = (lax.broadcasted_iota(jnp.int32, (1, T), 1) == 0).astype(jnp.float32)
        m_full = jnp.concatenate([row0, m], axis=0)
        mw_ref[...] = lax.dot_general(m_full, w_ref[...],
                                      (((1,), (0,)), ((), ())),
                                      preferred_element_type=jnp.float32) * SCALE

    o_ref[...] = lax.dot_general(x_ref[...], mw_ref[...],
                                 (((1,), (1,)), ((), ())),
                                 preferred_element_type=jnp.float32)


@jax.jit
def kernel(inputs, weight, feature_table):
    B = inputs.shape[0]
    blk = 8192
    grid = (B // blk,)
    return pl.pallas_call(
        _body,
        grid=grid,
        in_specs=[
            pl.BlockSpec((blk, E), lambda i: (i, 0)),
            pl.BlockSpec((T, E), lambda i: (0, 0)),
            pl.BlockSpec((V, F), lambda i: (0, 0)),
        ],
        out_specs=pl.BlockSpec((blk, V + 1), lambda i: (i, 0)),
        out_shape=jax.ShapeDtypeStruct((B, V + 1), jnp.float32),
        scratch_shapes=[pltpu.VMEM((V + 1, E), jnp.float32)],
    )(inputs, weight, feature_table)
